# Initial kernel scaffold; baseline (speedup 1.0000x reference)
import functools

import jax
import jax.numpy as jnp
from jax import lax
from jax.experimental import pallas as pl
from jax.experimental.pallas import tpu as pltpu
from jax.experimental.pallas import tpu_sc as plsc

NUM_HEROES = 1000000
EMBED_DIM = 32
BATCH = 16384
HIST = 20
TOTAL = BATCH * HIST

_info = plsc.get_sparse_core_info()
NC, NS = _info.num_cores, _info.num_subcores
NW = NC * NS
PER_W = TOTAL // NW
CHUNK = 2048
NCHUNK = PER_W // CHUNK

_mesh = plsc.VectorSubcoreMesh(core_axis_name="c", subcore_axis_name="s")


@functools.partial(
    pl.kernel,
    mesh=_mesh,
    out_type=jax.ShapeDtypeStruct((TOTAL, EMBED_DIM), jnp.float32),
    scratch_types=[
        pltpu.VMEM((CHUNK,), jnp.int32),
        pltpu.VMEM((CHUNK, EMBED_DIM), jnp.float32),
        pltpu.SemaphoreType.DMA,
    ],
)
def _gather(idx_hbm, table_hbm, out_hbm, idx_v, rows_v, sem):
    wid = lax.axis_index("s") * NC + lax.axis_index("c")
    base = wid * PER_W

    def body(g, carry):
        off = base + g * CHUNK
        pltpu.sync_copy(idx_hbm.at[pl.ds(off, CHUNK)], idx_v)
        pltpu.async_copy(table_hbm.at[idx_v], rows_v, sem).wait()
        pltpu.sync_copy(rows_v, out_hbm.at[pl.ds(off, CHUNK)])
        return carry

    lax.fori_loop(0, NCHUNK, body, 0)


def kernel(hero_ids, table):
    flat = hero_ids.reshape(TOTAL).astype(jnp.int32)
    out = _gather(flat, table)
    return out.reshape(BATCH, HIST, EMBED_DIM)


# speedup vs baseline: 1.5062x; 1.5062x over previous
import functools

import jax
import jax.numpy as jnp
from jax import lax
from jax.experimental import pallas as pl
from jax.experimental.pallas import tpu as pltpu
from jax.experimental.pallas import tpu_sc as plsc

NUM_HEROES = 1000000
EMBED_DIM = 32
BATCH = 16384
HIST = 20
TOTAL = BATCH * HIST

_info = plsc.get_sparse_core_info()
NC, NS = _info.num_cores, _info.num_subcores
NW = NC * NS
PER_W = TOTAL // NW
CHUNK = 2048
NCHUNK = PER_W // CHUNK

_mesh = plsc.VectorSubcoreMesh(core_axis_name="c", subcore_axis_name="s")


@functools.partial(
    pl.kernel,
    mesh=_mesh,
    out_type=jax.ShapeDtypeStruct((TOTAL, EMBED_DIM), jnp.float32),
    scratch_types=[
        pltpu.VMEM((CHUNK,), jnp.int32),
        pltpu.VMEM((CHUNK, EMBED_DIM), jnp.float32),
        pltpu.SemaphoreType.DMA,
    ],
    compiler_params=pltpu.CompilerParams(use_tc_tiling_on_sc=False),
)
def _gather(idx_hbm, table_hbm, out_hbm, idx_v, rows_v, sem):
    wid = lax.axis_index("s") * NC + lax.axis_index("c")
    base = wid * PER_W

    def body(g, carry):
        off = base + g * CHUNK
        pltpu.sync_copy(idx_hbm.at[pl.ds(off, CHUNK)], idx_v)
        pltpu.async_copy(table_hbm.at[idx_v], rows_v, sem).wait()
        pltpu.sync_copy(rows_v, out_hbm.at[pl.ds(off, CHUNK)])
        return carry

    lax.fori_loop(0, NCHUNK, body, 0)


def kernel(hero_ids, table):
    flat = hero_ids.reshape(TOTAL).astype(jnp.int32)
    out = _gather(flat, table)
    return out.reshape(BATCH, HIST, EMBED_DIM)


# speedup vs baseline: 1.5094x; 1.0022x over previous
import functools

import jax
import jax.numpy as jnp
from jax import lax
from jax.experimental import pallas as pl
from jax.experimental.pallas import tpu as pltpu
from jax.experimental.pallas import tpu_sc as plsc

NUM_HEROES = 1000000
EMBED_DIM = 32
BATCH = 16384
HIST = 20
TOTAL = BATCH * HIST

_info = plsc.get_sparse_core_info()
NC, NS = _info.num_cores, _info.num_subcores
NW = NC * NS
PER_W = TOTAL // NW
CHUNK = 1280
NCHUNK = PER_W // CHUNK
NB = 2

_mesh = plsc.VectorSubcoreMesh(core_axis_name="c", subcore_axis_name="s")


@functools.partial(
    pl.kernel,
    mesh=_mesh,
    out_type=jax.ShapeDtypeStruct((TOTAL, EMBED_DIM), jnp.float32),
    scratch_types=[
        pltpu.VMEM((NB, CHUNK), jnp.int32),
        pltpu.VMEM((NB, CHUNK, EMBED_DIM), jnp.float32),
        [pltpu.SemaphoreType.DMA] * NB,
        [pltpu.SemaphoreType.DMA] * NB,
    ],
    compiler_params=pltpu.CompilerParams(use_tc_tiling_on_sc=False),
)
def _gather(idx_hbm, table_hbm, out_hbm, idx_v, rows_v, gsems, osems):
    wid = lax.axis_index("s") * NC + lax.axis_index("c")
    base = wid * PER_W

    gathers = [None] * NCHUNK
    writes = [None] * NCHUNK
    for g in range(NCHUNK):
        b = g % NB
        off = base + g * CHUNK
        if g >= NB:
            writes[g - NB].wait()
        pltpu.sync_copy(idx_hbm.at[pl.ds(off, CHUNK)], idx_v.at[b])
        gathers[g] = pltpu.async_copy(table_hbm.at[idx_v.at[b]], rows_v.at[b], gsems[b])
        if g >= 1:
            pb = (g - 1) % NB
            gathers[g - 1].wait()
            writes[g - 1] = pltpu.async_copy(
                rows_v.at[pb], out_hbm.at[pl.ds(base + (g - 1) * CHUNK, CHUNK)], osems[pb]
            )
    last = NCHUNK - 1
    gathers[last].wait()
    writes[last] = pltpu.async_copy(
        rows_v.at[last % NB], out_hbm.at[pl.ds(base + last * CHUNK, CHUNK)], osems[last % NB]
    )
    for g in range(max(0, NCHUNK - NB), NCHUNK):
        writes[g].wait()


def kernel(hero_ids, table):
    flat = hero_ids.reshape(TOTAL).astype(jnp.int32)
    out = _gather(flat, table)
    return out.reshape(BATCH, HIST, EMBED_DIM)


# speedup vs baseline: 1.6017x; 1.0611x over previous
import functools

import jax
import jax.numpy as jnp
from jax import lax
from jax.experimental import pallas as pl
from jax.experimental.pallas import tpu as pltpu
from jax.experimental.pallas import tpu_sc as plsc

NUM_HEROES = 1000000
EMBED_DIM = 32
BATCH = 16384
HIST = 20
TOTAL = BATCH * HIST

_info = plsc.get_sparse_core_info()
NC, NS = _info.num_cores, _info.num_subcores
NW = NC * NS
PER_W = TOTAL // NW
CHUNK = 1280
NCHUNK = PER_W // CHUNK
NB = 2

_mesh = plsc.VectorSubcoreMesh(core_axis_name="c", subcore_axis_name="s")


@functools.partial(
    pl.kernel,
    mesh=_mesh,
    out_type=jax.ShapeDtypeStruct((TOTAL, EMBED_DIM), jnp.float32),
    scratch_types=[
        pltpu.VMEM((NB, CHUNK), jnp.int32),
        pltpu.VMEM((NB, CHUNK, EMBED_DIM), jnp.float32),
        [pltpu.SemaphoreType.DMA] * NB,
        [pltpu.SemaphoreType.DMA] * NB,
    ],
    compiler_params=pltpu.CompilerParams(use_tc_tiling_on_sc=False),
)
def _gather(idx_hbm, table_hbm, out_hbm, idx_v, rows_v, gsems, osems):
    wid = lax.axis_index("s") * NC + lax.axis_index("c")
    base = wid * PER_W

    gathers = [None] * NCHUNK
    writes = [None] * NCHUNK
    for g in range(NCHUNK):
        b = g % NB
        off = base + g * CHUNK
        if g >= NB:
            writes[g - NB].wait()
        pltpu.sync_copy(idx_hbm.at[pl.ds(off, CHUNK)], idx_v.at[b])
        gathers[g] = pltpu.async_copy(table_hbm.at[idx_v.at[b]], rows_v.at[b], gsems[b])
        if g >= 1:
            pb = (g - 1) % NB
            gathers[g - 1].wait()
            writes[g - 1] = pltpu.async_copy(
                rows_v.at[pb], out_hbm.at[pl.ds(base + (g - 1) * CHUNK, CHUNK)], osems[pb]
            )
    last = NCHUNK - 1
    gathers[last].wait()
    writes[last] = pltpu.async_copy(
        rows_v.at[last % NB], out_hbm.at[pl.ds(base + last * CHUNK, CHUNK)], osems[last % NB]
    )
    for g in range(max(0, NCHUNK - NB), NCHUNK):
        writes[g].wait()


def kernel(hero_ids, table):
    ids_hm = hero_ids.T.reshape(TOTAL).astype(jnp.int32)
    out_hm = _gather(ids_hm, table)
    return out_hm.reshape(HIST, BATCH, EMBED_DIM).transpose(1, 0, 2)
